# baseline (speedup 1.0000x reference)
import functools

import jax
import jax.numpy as jnp
from jax import lax
from jax.experimental import pallas as pl
from jax.experimental.pallas import tpu as pltpu
from jax.experimental.pallas import tpu_sc as plsc

N = 10000
E = 320000
D = 128
NC = 2
NS = 16
NW = NC * NS
EPW = E // NW
ROWS = 125
CHUNK = 80
NPW = N // NS
ECHUNK = 128
NCH = 78
IROWS = 26

_mesh = plsc.VectorSubcoreMesh(core_axis_name="c", subcore_axis_name="s")
_f32 = jnp.float32
_bf16 = jnp.bfloat16


@functools.partial(
    pl.kernel,
    out_type=[jax.ShapeDtypeStruct((NC * N,), _f32),
              jax.ShapeDtypeStruct((NC * N,), _f32)],
    mesh=_mesh,
    compiler_params=pltpu.CompilerParams(use_tc_tiling_on_sc=False),
    scratch_types=[
        pltpu.VMEM_SHARED((N,), _f32),
        pltpu.VMEM_SHARED((N,), _f32),
        pltpu.VMEM((ROWS, CHUNK), jnp.int32),
        pltpu.VMEM((ROWS, CHUNK), jnp.int32),
        pltpu.VMEM((CHUNK,), _f32),
        pltpu.VMEM((1000,), _f32),
        pltpu.SemaphoreType.DMA,
        pltpu.SemaphoreType.DMA,
        pltpu.SemaphoreType.DMA,
        pltpu.SemaphoreType.DMA,
        pltpu.SemaphoreType.DMA,
        pltpu.SemaphoreType.DMA,
        pltpu.SemaphoreType.DMA,
        pltpu.SemaphoreType.DMA,
    ],
)
def _deg_kernel(src_m, dst_m, ones_hbm, zeros_hbm, deg_s_out, deg_d_out,
                acc_s, acc_d, sidx, didx, ones_v, zbuf,
                d0, d1, d2, d3, d4, d5, d6, d7):
    c = lax.axis_index("c")
    s = lax.axis_index("s")
    wid = c * NS + s
    pltpu.sync_copy(zeros_hbm, zbuf)

    @pl.when(s < 10)
    def _zero():
        pltpu.sync_copy(zbuf, acc_s.at[pl.ds(pl.multiple_of(s * 1000, 8), 1000)])
        pltpu.sync_copy(zbuf, acc_d.at[pl.ds(pl.multiple_of(s * 1000, 8), 1000)])

    pltpu.sync_copy(ones_hbm, ones_v)
    pltpu.sync_copy(src_m.at[wid], sidx)
    pltpu.sync_copy(dst_m.at[wid], didx)
    plsc.subcore_barrier()
    dsems = (d0, d1, d2, d3, d4, d5, d6, d7)
    pend = [None] * 8
    for j in range(2 * ROWS):
        q = j % 8
        if pend[q] is not None:
            pend[q].wait()
        if j % 2 == 0:
            pend[q] = pltpu.async_copy(ones_v, acc_s.at[sidx.at[j // 2]],
                                       dsems[q], add=True)
        else:
            pend[q] = pltpu.async_copy(ones_v, acc_d.at[didx.at[j // 2]],
                                       dsems[q], add=True)
    for q in range(8):
        if pend[q] is not None:
            pend[q].wait()
    plsc.subcore_barrier()

    @pl.when(s < 10)
    def _dump():
        pltpu.sync_copy(acc_s.at[pl.ds(pl.multiple_of(s * 1000, 8), 1000)], zbuf)
        pltpu.sync_copy(zbuf, deg_s_out.at[pl.ds(pl.multiple_of(c * N + s * 1000, 8), 1000)])
        pltpu.sync_copy(acc_d.at[pl.ds(pl.multiple_of(s * 1000, 8), 1000)], zbuf)
        pltpu.sync_copy(zbuf, deg_d_out.at[pl.ds(pl.multiple_of(c * N + s * 1000, 8), 1000)])


@functools.partial(
    pl.kernel,
    out_type=jax.ShapeDtypeStruct((NC * N, D), _bf16),
    mesh=_mesh,
    compiler_params=pltpu.CompilerParams(use_tc_tiling_on_sc=False),
    scratch_types=[
        pltpu.VMEM_SHARED((N, D), _bf16),
        pltpu.VMEM((2, IROWS, ECHUNK), jnp.int32),
        pltpu.VMEM((2, IROWS, ECHUNK), jnp.int32),
        pltpu.VMEM((ECHUNK, D), _bf16),
        pltpu.VMEM((ECHUNK, D), _bf16),
        pltpu.VMEM((ECHUNK, D), _bf16),
        pltpu.VMEM((ECHUNK, D), _bf16),
        pltpu.VMEM((ECHUNK, D), _bf16),
        pltpu.VMEM((ECHUNK, D), _bf16),
        pltpu.VMEM((ECHUNK, D), _bf16),
        pltpu.SemaphoreType.DMA,
        pltpu.SemaphoreType.DMA,
        pltpu.SemaphoreType.DMA,
        pltpu.SemaphoreType.DMA,
        pltpu.SemaphoreType.DMA,
        pltpu.SemaphoreType.DMA,
        pltpu.SemaphoreType.DMA,
        pltpu.SemaphoreType.DMA,
        pltpu.SemaphoreType.DMA,
        pltpu.SemaphoreType.DMA,
        pltpu.SemaphoreType.DMA,
        pltpu.SemaphoreType.DMA,
    ],
)
def _agg_kernel(table, src_m, dst_m, zeros_hbm, out,
                acc, sidx, didx, b0, b1, b2, b3, b4, b5, dbuf,
                g0, g1, g2, g3, g4, g5, s0, s1, s2, s3, s4, s5):
    c = lax.axis_index("c")
    s = lax.axis_index("s")
    wid = c * NS + s
    accv = acc
    bufs = (b0, b1, b2, b3, b4, b5)
    gsems = (g0, g1, g2, g3, g4, g5)
    ssems = (s0, s1, s2, s3, s4, s5)
    RING = 6
    LAG = 3
    pltpu.sync_copy(zeros_hbm, dbuf)
    for k in range(5):
        chunk = s + k * NS

        @pl.when(chunk < 78)
        def _zero(chunk=chunk):
            pltpu.sync_copy(dbuf, acc.at[pl.ds(pl.multiple_of(chunk * ECHUNK, 8), ECHUNK)])

        @pl.when(chunk == 78)
        def _zero_tail(chunk=chunk):
            pltpu.sync_copy(dbuf.at[pl.ds(0, 16)],
                            acc.at[pl.ds(pl.multiple_of(chunk * ECHUNK, 8), 16)])

    plsc.subcore_barrier()
    base = wid * NCH
    pend_g = [None] * 6
    pend_s = [None] * 6
    nstage = NCH // IROWS
    for k in range(nstage):
        p = k % 2
        pltpu.sync_copy(src_m.at[pl.ds(base + k * IROWS, IROWS)], sidx.at[p])
        pltpu.sync_copy(dst_m.at[pl.ds(base + k * IROWS, IROWS)], didx.at[p])
        for j in range(IROWS):
            g = k * IROWS + j
            b = g % RING
            if pend_s[b] is not None:
                pend_s[b].wait()
                pend_s[b] = None
            pend_g[b] = pltpu.async_copy(table.at[sidx.at[p, j]], bufs[b], gsems[b])
            g2_ = g - LAG
            if g2_ >= 0:
                b2_ = g2_ % RING
                j2 = g2_ % IROWS
                p2 = (g2_ // IROWS) % 2
                pend_g[b2_].wait()
                pend_s[b2_] = pltpu.async_copy(
                    bufs[b2_], accv.at[didx.at[p2, j2]], ssems[b2_], add=True)
    for g2_ in range(NCH - LAG, NCH):
        b2_ = g2_ % RING
        j2 = g2_ % IROWS
        p2 = (g2_ // IROWS) % 2
        pend_g[b2_].wait()
        pend_s[b2_] = pltpu.async_copy(
            bufs[b2_], accv.at[didx.at[p2, j2]], ssems[b2_], add=True)
    for b in range(6):
        if pend_s[b] is not None:
            pend_s[b].wait()
    @pl.when(wid < 4)
    def _tail():
        pltpu.sync_copy(src_m.at[NW * NCH + wid], sidx.at[0, 0])
        pltpu.sync_copy(dst_m.at[NW * NCH + wid], didx.at[0, 0])
        pltpu.async_copy(table.at[sidx.at[0, 0]], b0, g0).wait()
        pltpu.async_copy(b0, accv.at[didx.at[0, 0]], s0, add=True).wait()

    plsc.subcore_barrier()
    for k in range(5):
        chunk = s + k * NS

        @pl.when(chunk < 78)
        def _dump(chunk=chunk):
            pltpu.sync_copy(acc.at[pl.ds(pl.multiple_of(chunk * ECHUNK, 8), ECHUNK)], dbuf)
            pltpu.sync_copy(dbuf, out.at[pl.ds(pl.multiple_of(c * N + chunk * ECHUNK, 8), ECHUNK)])

        @pl.when(chunk == 78)
        def _dump_tail(chunk=chunk):
            pltpu.sync_copy(acc.at[pl.ds(pl.multiple_of(chunk * ECHUNK, 8), 16)], dbuf.at[pl.ds(0, 16)])
            pltpu.sync_copy(dbuf.at[pl.ds(0, 16)],
                            out.at[pl.ds(pl.multiple_of(c * N + chunk * ECHUNK, 8), 16)])


_BLK = 1000
_GRID = N // _BLK


def _norm(deg_ref):
    d = deg_ref[0] + deg_ref[1]
    return lax.rsqrt(jnp.maximum(d, 1.0))


def _xw_body(x_ref, w1_ref, xw_ref):
    xw_ref[...] = jnp.dot(x_ref[...], w1_ref[...], preferred_element_type=_f32)


def _pre_body(xw_ref, ds_ref, t1_ref):
    t1_ref[...] = (xw_ref[...] * _norm(ds_ref)).astype(_bf16)


def _mid_body(p0_ref, p1_ref, dd_ref, ds_ref, b_ref, w_ref, h_ref, t_ref):
    agg = p0_ref[...].astype(_f32) + p1_ref[...].astype(_f32)
    y = agg * _norm(dd_ref) + b_ref[...]
    h = jnp.where(y > 0, y, jnp.exp(jnp.minimum(y, 0.0)) - 1.0)
    h_ref[...] = h
    hw = jnp.dot(h, w_ref[...], preferred_element_type=_f32)
    t_ref[...] = (hw * _norm(ds_ref)).astype(_bf16)


def _last_body(p0_ref, p1_ref, dd_ref, b_ref, h_ref):
    agg = p0_ref[...].astype(_f32) + p1_ref[...].astype(_f32)
    y = agg * _norm(dd_ref) + b_ref[...]
    m = jnp.max(y, axis=1, keepdims=True)
    e = jnp.exp(y - m)
    h_ref[...] = e / jnp.sum(e, axis=1, keepdims=True)


def _row_spec():
    return pl.BlockSpec((_BLK, D), lambda i: (i, 0))


def _deg_spec():
    return pl.BlockSpec((2, _BLK, 1), lambda i: (0, i, 0))


def _tc_xw(x, w1):
    return pl.pallas_call(
        _xw_body,
        grid=(_GRID,),
        in_specs=[
            _row_spec(),
            pl.BlockSpec((D, D), lambda i: (0, 0)),
        ],
        out_specs=_row_spec(),
        out_shape=jax.ShapeDtypeStruct((N, D), _f32),
    )(x, w1)


def _tc_pre(xw, deg_s):
    return pl.pallas_call(
        _pre_body,
        grid=(_GRID,),
        in_specs=[
            _row_spec(),
            _deg_spec(),
        ],
        out_specs=_row_spec(),
        out_shape=jax.ShapeDtypeStruct((N, D), _bf16),
    )(xw, deg_s)


def _tc_mid(p, deg_d, deg_s, b, w_next):
    return pl.pallas_call(
        _mid_body,
        grid=(_GRID,),
        in_specs=[
            pl.BlockSpec((_BLK, D), lambda i: (i, 0)),
            pl.BlockSpec((_BLK, D), lambda i: (_GRID + i, 0)),
            _deg_spec(),
            _deg_spec(),
            pl.BlockSpec((1, D), lambda i: (0, 0)),
            pl.BlockSpec((D, D), lambda i: (0, 0)),
        ],
        out_specs=[_row_spec(), _row_spec()],
        out_shape=[jax.ShapeDtypeStruct((N, D), _f32),
                   jax.ShapeDtypeStruct((N, D), _bf16)],
    )(p, p, deg_d, deg_s, b, w_next)


def _tc_last(p, deg_d, b):
    return pl.pallas_call(
        _last_body,
        grid=(_GRID,),
        in_specs=[
            pl.BlockSpec((_BLK, D), lambda i: (i, 0)),
            pl.BlockSpec((_BLK, D), lambda i: (_GRID + i, 0)),
            _deg_spec(),
            pl.BlockSpec((1, D), lambda i: (0, 0)),
        ],
        out_specs=_row_spec(),
        out_shape=jax.ShapeDtypeStruct((N, D), _f32),
    )(p, p, deg_d, b)


def kernel(x, edge_index, W1, b1, W2, b2, W3, b3):
    src = edge_index[0].astype(jnp.int32).reshape(NW, ROWS, CHUNK)
    dst = edge_index[1].astype(jnp.int32).reshape(NW, ROWS, CHUNK)
    src_e = edge_index[0].astype(jnp.int32).reshape(E // ECHUNK, ECHUNK)
    dst_e = edge_index[1].astype(jnp.int32).reshape(E // ECHUNK, ECHUNK)
    ones_c = jnp.ones((CHUNK,), _f32)
    zeros_1k = jnp.zeros((1000,), _f32)
    zeros_rows = jnp.zeros((ECHUNK, D), _bf16)

    xw1 = _tc_xw(x, W1)
    deg_s, deg_d = _deg_kernel(src, dst, ones_c, zeros_1k)
    deg_s = deg_s.reshape(NC, N, 1)
    deg_d = deg_d.reshape(NC, N, 1)
    table1 = _tc_pre(xw1, deg_s)

    p1 = _agg_kernel(table1, src_e, dst_e, zeros_rows)
    h1, table2 = _tc_mid(p1, deg_d, deg_s, b1.reshape(1, D), W2)

    p2 = _agg_kernel(table2, src_e, dst_e, zeros_rows)
    h2, table3 = _tc_mid(p2, deg_d, deg_s, b2.reshape(1, D), W3)

    p3 = _agg_kernel(table3, src_e, dst_e, zeros_rows)
    h3 = _tc_last(p3, deg_d, b3.reshape(1, D))

    return (h1.reshape(1, N, D), h2.reshape(1, N, D), h3.reshape(1, N, D))


# speedup vs baseline: 1.0098x; 1.0098x over previous
import functools

import jax
import jax.numpy as jnp
from jax import lax
from jax.experimental import pallas as pl
from jax.experimental.pallas import tpu as pltpu
from jax.experimental.pallas import tpu_sc as plsc

N = 10000
E = 320000
D = 128
NC = 2
NS = 16
NW = NC * NS
EPW = E // NW
ROWS = 125
CHUNK = 80
NPW = N // NS
ECHUNK = 128
NCH = 78
IROWS = 26

_mesh = plsc.VectorSubcoreMesh(core_axis_name="c", subcore_axis_name="s")
_f32 = jnp.float32
_bf16 = jnp.bfloat16


@functools.partial(
    pl.kernel,
    out_type=[jax.ShapeDtypeStruct((NC * N,), _f32),
              jax.ShapeDtypeStruct((NC * N,), _f32)],
    mesh=_mesh,
    compiler_params=pltpu.CompilerParams(use_tc_tiling_on_sc=False),
    scratch_types=[
        pltpu.VMEM_SHARED((N,), _f32),
        pltpu.VMEM_SHARED((N,), _f32),
        pltpu.VMEM((ROWS, CHUNK), jnp.int32),
        pltpu.VMEM((ROWS, CHUNK), jnp.int32),
        pltpu.VMEM((CHUNK,), _f32),
        pltpu.VMEM((1000,), _f32),
        pltpu.SemaphoreType.DMA,
        pltpu.SemaphoreType.DMA,
        pltpu.SemaphoreType.DMA,
        pltpu.SemaphoreType.DMA,
        pltpu.SemaphoreType.DMA,
        pltpu.SemaphoreType.DMA,
        pltpu.SemaphoreType.DMA,
        pltpu.SemaphoreType.DMA,
    ],
)
def _deg_kernel(src_m, dst_m, ones_hbm, zeros_hbm, deg_s_out, deg_d_out,
                acc_s, acc_d, sidx, didx, ones_v, zbuf,
                d0, d1, d2, d3, d4, d5, d6, d7):
    c = lax.axis_index("c")
    s = lax.axis_index("s")
    wid = c * NS + s
    pltpu.sync_copy(zeros_hbm, zbuf)

    @pl.when(s < 10)
    def _zero():
        pltpu.sync_copy(zbuf, acc_s.at[pl.ds(pl.multiple_of(s * 1000, 8), 1000)])
        pltpu.sync_copy(zbuf, acc_d.at[pl.ds(pl.multiple_of(s * 1000, 8), 1000)])

    pltpu.sync_copy(ones_hbm, ones_v)
    pltpu.sync_copy(src_m.at[wid], sidx)
    pltpu.sync_copy(dst_m.at[wid], didx)
    plsc.subcore_barrier()
    dsems = (d0, d1, d2, d3, d4, d5, d6, d7)
    pend = [None] * 8
    for j in range(2 * ROWS):
        q = j % 8
        if pend[q] is not None:
            pend[q].wait()
        if j % 2 == 0:
            pend[q] = pltpu.async_copy(ones_v, acc_s.at[sidx.at[j // 2]],
                                       dsems[q], add=True)
        else:
            pend[q] = pltpu.async_copy(ones_v, acc_d.at[didx.at[j // 2]],
                                       dsems[q], add=True)
    for q in range(8):
        if pend[q] is not None:
            pend[q].wait()
    plsc.subcore_barrier()

    @pl.when(s < 10)
    def _dump():
        pltpu.sync_copy(acc_s.at[pl.ds(pl.multiple_of(s * 1000, 8), 1000)], zbuf)
        pltpu.sync_copy(zbuf, deg_s_out.at[pl.ds(pl.multiple_of(c * N + s * 1000, 8), 1000)])
        pltpu.sync_copy(acc_d.at[pl.ds(pl.multiple_of(s * 1000, 8), 1000)], zbuf)
        pltpu.sync_copy(zbuf, deg_d_out.at[pl.ds(pl.multiple_of(c * N + s * 1000, 8), 1000)])


@functools.partial(
    pl.kernel,
    out_type=jax.ShapeDtypeStruct((NC * N, D), _bf16),
    mesh=_mesh,
    compiler_params=pltpu.CompilerParams(use_tc_tiling_on_sc=False),
    scratch_types=[
        pltpu.VMEM_SHARED((N, D), _bf16),
        pltpu.VMEM((2, IROWS, ECHUNK), jnp.int32),
        pltpu.VMEM((2, IROWS, ECHUNK), jnp.int32),
        pltpu.VMEM((ECHUNK, D), _bf16),
        pltpu.VMEM((ECHUNK, D), _bf16),
        pltpu.VMEM((ECHUNK, D), _bf16),
        pltpu.VMEM((ECHUNK, D), _bf16),
        pltpu.VMEM((ECHUNK, D), _bf16),
        pltpu.VMEM((ECHUNK, D), _bf16),
        pltpu.VMEM((ECHUNK, D), _bf16),
        pltpu.SemaphoreType.DMA,
        pltpu.SemaphoreType.DMA,
        pltpu.SemaphoreType.DMA,
        pltpu.SemaphoreType.DMA,
        pltpu.SemaphoreType.DMA,
        pltpu.SemaphoreType.DMA,
        pltpu.SemaphoreType.DMA,
        pltpu.SemaphoreType.DMA,
        pltpu.SemaphoreType.DMA,
        pltpu.SemaphoreType.DMA,
        pltpu.SemaphoreType.DMA,
        pltpu.SemaphoreType.DMA,
    ],
)
def _agg_kernel(table, src_m, dst_m, zeros_hbm, out,
                acc, sidx, didx, b0, b1, b2, b3, b4, b5, dbuf,
                g0, g1, g2, g3, g4, g5, s0, s1, s2, s3, s4, s5):
    c = lax.axis_index("c")
    s = lax.axis_index("s")
    wid = c * NS + s
    accv = acc
    bufs = (b0, b1, b2, b3, b4, b5)
    gsems = (g0, g1, g2, g3, g4, g5)
    ssems = (s0, s1, s2, s3, s4, s5)
    RING = 6
    LAG = 3
    pltpu.sync_copy(zeros_hbm, dbuf)
    for k in range(5):
        chunk = s + k * NS

        @pl.when(chunk < 78)
        def _zero(chunk=chunk):
            pltpu.sync_copy(dbuf, acc.at[pl.ds(pl.multiple_of(chunk * ECHUNK, 8), ECHUNK)])

        @pl.when(chunk == 78)
        def _zero_tail(chunk=chunk):
            pltpu.sync_copy(dbuf.at[pl.ds(0, 16)],
                            acc.at[pl.ds(pl.multiple_of(chunk * ECHUNK, 8), 16)])

    plsc.subcore_barrier()
    base = wid * NCH
    pend_g = [None] * 6
    pend_s = [None] * 6
    nstage = NCH // IROWS
    for k in range(nstage):
        p = k % 2
        pltpu.sync_copy(src_m.at[pl.ds(base + k * IROWS, IROWS)], sidx.at[p])
        pltpu.sync_copy(dst_m.at[pl.ds(base + k * IROWS, IROWS)], didx.at[p])
        for j in range(IROWS):
            g = k * IROWS + j
            b = g % RING
            if pend_s[b] is not None:
                pend_s[b].wait()
                pend_s[b] = None
            pend_g[b] = pltpu.async_copy(table.at[sidx.at[p, j]], bufs[b], gsems[b])
            g2_ = g - LAG
            if g2_ >= 0:
                b2_ = g2_ % RING
                j2 = g2_ % IROWS
                p2 = (g2_ // IROWS) % 2
                pend_g[b2_].wait()
                pend_s[b2_] = pltpu.async_copy(
                    bufs[b2_], accv.at[didx.at[p2, j2]], ssems[b2_], add=True)
    for g2_ in range(NCH - LAG, NCH):
        b2_ = g2_ % RING
        j2 = g2_ % IROWS
        p2 = (g2_ // IROWS) % 2
        pend_g[b2_].wait()
        pend_s[b2_] = pltpu.async_copy(
            bufs[b2_], accv.at[didx.at[p2, j2]], ssems[b2_], add=True)
    for b in range(6):
        if pend_s[b] is not None:
            pend_s[b].wait()
    @pl.when(wid < 4)
    def _tail():
        pltpu.sync_copy(src_m.at[NW * NCH + wid], sidx.at[0, 0])
        pltpu.sync_copy(dst_m.at[NW * NCH + wid], didx.at[0, 0])
        pltpu.async_copy(table.at[sidx.at[0, 0]], b0, g0).wait()
        pltpu.async_copy(b0, accv.at[didx.at[0, 0]], s0, add=True).wait()

    plsc.subcore_barrier()
    for k in range(5):
        chunk = s + k * NS

        @pl.when(chunk < 78)
        def _dump(chunk=chunk):
            pltpu.sync_copy(acc.at[pl.ds(pl.multiple_of(chunk * ECHUNK, 8), ECHUNK)], dbuf)
            pltpu.sync_copy(dbuf, out.at[pl.ds(pl.multiple_of(c * N + chunk * ECHUNK, 8), ECHUNK)])

        @pl.when(chunk == 78)
        def _dump_tail(chunk=chunk):
            pltpu.sync_copy(acc.at[pl.ds(pl.multiple_of(chunk * ECHUNK, 8), 16)], dbuf.at[pl.ds(0, 16)])
            pltpu.sync_copy(dbuf.at[pl.ds(0, 16)],
                            out.at[pl.ds(pl.multiple_of(c * N + chunk * ECHUNK, 8), 16)])


_BLK = 1000
_GRID = N // _BLK


def _norm(deg_ref):
    d = deg_ref[0] + deg_ref[1]
    return lax.rsqrt(jnp.maximum(d, 1.0))


def _pre_body(x_ref, w1_ref, ds_ref, t1_ref):
    xw = jnp.dot(x_ref[...], w1_ref[...], preferred_element_type=_f32)
    t1_ref[...] = (xw * _norm(ds_ref)).astype(_bf16)


def _mid_body(p_ref, dd_ref, ds_ref, b_ref, w_ref, h_ref, t_ref):
    agg = p_ref[0].astype(_f32) + p_ref[1].astype(_f32)
    y = agg * _norm(dd_ref) + b_ref[...]
    h = jnp.where(y > 0, y, jnp.exp(jnp.minimum(y, 0.0)) - 1.0)
    h_ref[...] = h
    hw = jnp.dot(h, w_ref[...], preferred_element_type=_f32)
    t_ref[...] = (hw * _norm(ds_ref)).astype(_bf16)


def _last_body(p_ref, dd_ref, b_ref, h_ref):
    agg = p_ref[0].astype(_f32) + p_ref[1].astype(_f32)
    y = agg * _norm(dd_ref) + b_ref[...]
    m = jnp.max(y, axis=1, keepdims=True)
    e = jnp.exp(y - m)
    h_ref[...] = e / jnp.sum(e, axis=1, keepdims=True)


def _row_spec():
    return pl.BlockSpec((_BLK, D), lambda i: (i, 0))


def _deg_spec():
    return pl.BlockSpec((2, _BLK, 1), lambda i: (0, i, 0))


def _tc_pre(x, w1, deg_s):
    return pl.pallas_call(
        _pre_body,
        grid=(_GRID,),
        in_specs=[
            _row_spec(),
            pl.BlockSpec((D, D), lambda i: (0, 0)),
            _deg_spec(),
        ],
        out_specs=_row_spec(),
        out_shape=jax.ShapeDtypeStruct((N, D), _bf16),
    )(x, w1, deg_s)


def _tc_mid(p, deg_d, deg_s, b, w_next):
    return pl.pallas_call(
        _mid_body,
        grid=(_GRID,),
        in_specs=[
            pl.BlockSpec((2, _BLK, D), lambda i: (0, i, 0)),
            _deg_spec(),
            _deg_spec(),
            pl.BlockSpec((1, D), lambda i: (0, 0)),
            pl.BlockSpec((D, D), lambda i: (0, 0)),
        ],
        out_specs=[_row_spec(), _row_spec()],
        out_shape=[jax.ShapeDtypeStruct((N, D), _f32),
                   jax.ShapeDtypeStruct((N, D), _bf16)],
    )(p, deg_d, deg_s, b, w_next)


def _tc_last(p, deg_d, b):
    return pl.pallas_call(
        _last_body,
        grid=(_GRID,),
        in_specs=[
            pl.BlockSpec((2, _BLK, D), lambda i: (0, i, 0)),
            _deg_spec(),
            pl.BlockSpec((1, D), lambda i: (0, 0)),
        ],
        out_specs=_row_spec(),
        out_shape=jax.ShapeDtypeStruct((N, D), _f32),
    )(p, deg_d, b)


def kernel(x, edge_index, W1, b1, W2, b2, W3, b3):
    src = edge_index[0].astype(jnp.int32).reshape(NW, ROWS, CHUNK)
    dst = edge_index[1].astype(jnp.int32).reshape(NW, ROWS, CHUNK)
    src_e = edge_index[0].astype(jnp.int32).reshape(E // ECHUNK, ECHUNK)
    dst_e = edge_index[1].astype(jnp.int32).reshape(E // ECHUNK, ECHUNK)
    ones_c = jnp.ones((CHUNK,), _f32)
    zeros_1k = jnp.zeros((1000,), _f32)
    zeros_rows = jnp.zeros((ECHUNK, D), _bf16)

    deg_s, deg_d = _deg_kernel(src, dst, ones_c, zeros_1k)
    deg_s = deg_s.reshape(NC, N, 1)
    deg_d = deg_d.reshape(NC, N, 1)
    table1 = _tc_pre(x, W1, deg_s)

    p1 = _agg_kernel(table1, src_e, dst_e, zeros_rows).reshape(NC, N, D)
    h1, table2 = _tc_mid(p1, deg_d, deg_s, b1.reshape(1, D), W2)

    p2 = _agg_kernel(table2, src_e, dst_e, zeros_rows).reshape(NC, N, D)
    h2, table3 = _tc_mid(p2, deg_d, deg_s, b2.reshape(1, D), W3)

    p3 = _agg_kernel(table3, src_e, dst_e, zeros_rows).reshape(NC, N, D)
    h3 = _tc_last(p3, deg_d, b3.reshape(1, D))

    return (h1.reshape(1, N, D), h2.reshape(1, N, D), h3.reshape(1, N, D))


# speedup vs baseline: 1.0193x; 1.0094x over previous
import functools

import jax
import jax.numpy as jnp
from jax import lax
from jax.experimental import pallas as pl
from jax.experimental.pallas import tpu as pltpu
from jax.experimental.pallas import tpu_sc as plsc

N = 10000
E = 320000
D = 128
NC = 2
NS = 16
NW = NC * NS
EPW = E // NW
ROWS = 125
CHUNK = 80
NPW = N // NS
ECHUNK = 128
NCH = 78
IROWS = 26

_mesh = plsc.VectorSubcoreMesh(core_axis_name="c", subcore_axis_name="s")
_f32 = jnp.float32
_bf16 = jnp.bfloat16


@functools.partial(
    pl.kernel,
    out_type=[jax.ShapeDtypeStruct((NC * N,), _f32),
              jax.ShapeDtypeStruct((NC * N,), _f32)],
    mesh=_mesh,
    compiler_params=pltpu.CompilerParams(use_tc_tiling_on_sc=False),
    scratch_types=[
        pltpu.VMEM_SHARED((N,), _f32),
        pltpu.VMEM_SHARED((N,), _f32),
        pltpu.VMEM((ROWS, CHUNK), jnp.int32),
        pltpu.VMEM((ROWS, CHUNK), jnp.int32),
        pltpu.VMEM((CHUNK,), _f32),
        pltpu.VMEM((1000,), _f32),
        pltpu.SemaphoreType.DMA,
        pltpu.SemaphoreType.DMA,
        pltpu.SemaphoreType.DMA,
        pltpu.SemaphoreType.DMA,
        pltpu.SemaphoreType.DMA,
        pltpu.SemaphoreType.DMA,
        pltpu.SemaphoreType.DMA,
        pltpu.SemaphoreType.DMA,
    ],
)
def _deg_kernel(src_m, dst_m, ones_hbm, zeros_hbm, deg_s_out, deg_d_out,
                acc_s, acc_d, sidx, didx, ones_v, zbuf,
                d0, d1, d2, d3, d4, d5, d6, d7):
    c = lax.axis_index("c")
    s = lax.axis_index("s")
    wid = c * NS + s
    pltpu.sync_copy(zeros_hbm, zbuf)

    @pl.when(s < 10)
    def _zero():
        pltpu.sync_copy(zbuf, acc_s.at[pl.ds(pl.multiple_of(s * 1000, 8), 1000)])
        pltpu.sync_copy(zbuf, acc_d.at[pl.ds(pl.multiple_of(s * 1000, 8), 1000)])

    pltpu.sync_copy(ones_hbm, ones_v)
    pltpu.sync_copy(src_m.at[wid], sidx)
    pltpu.sync_copy(dst_m.at[wid], didx)
    plsc.subcore_barrier()
    dsems = (d0, d1, d2, d3, d4, d5, d6, d7)
    pend = [None] * 8
    for j in range(2 * ROWS):
        q = j % 8
        if pend[q] is not None:
            pend[q].wait()
        if j % 2 == 0:
            pend[q] = pltpu.async_copy(ones_v, acc_s.at[sidx.at[j // 2]],
                                       dsems[q], add=True)
        else:
            pend[q] = pltpu.async_copy(ones_v, acc_d.at[didx.at[j // 2]],
                                       dsems[q], add=True)
    for q in range(8):
        if pend[q] is not None:
            pend[q].wait()
    plsc.subcore_barrier()

    @pl.when(s < 10)
    def _dump():
        pltpu.sync_copy(acc_s.at[pl.ds(pl.multiple_of(s * 1000, 8), 1000)], zbuf)
        pltpu.sync_copy(zbuf, deg_s_out.at[pl.ds(pl.multiple_of(c * N + s * 1000, 8), 1000)])
        pltpu.sync_copy(acc_d.at[pl.ds(pl.multiple_of(s * 1000, 8), 1000)], zbuf)
        pltpu.sync_copy(zbuf, deg_d_out.at[pl.ds(pl.multiple_of(c * N + s * 1000, 8), 1000)])


@functools.partial(
    pl.kernel,
    out_type=jax.ShapeDtypeStruct((NC * N, D), _bf16),
    mesh=_mesh,
    compiler_params=pltpu.CompilerParams(use_tc_tiling_on_sc=False),
    scratch_types=[
        pltpu.VMEM_SHARED((N, D), _bf16),
        pltpu.VMEM((NCH, ECHUNK), jnp.int32),
        pltpu.VMEM((NCH, ECHUNK), jnp.int32),
        pltpu.VMEM((ECHUNK, D), _bf16),
        pltpu.VMEM((ECHUNK, D), _bf16),
        pltpu.VMEM((ECHUNK, D), _bf16),
        pltpu.VMEM((ECHUNK, D), _bf16),
        pltpu.VMEM((ECHUNK, D), _bf16),
        pltpu.VMEM((ECHUNK, D), _bf16),
        pltpu.VMEM((ECHUNK, D), _bf16),
        pltpu.SemaphoreType.DMA,
        pltpu.SemaphoreType.DMA,
        pltpu.SemaphoreType.DMA,
        pltpu.SemaphoreType.DMA,
        pltpu.SemaphoreType.DMA,
        pltpu.SemaphoreType.DMA,
        pltpu.SemaphoreType.DMA,
        pltpu.SemaphoreType.DMA,
        pltpu.SemaphoreType.DMA,
        pltpu.SemaphoreType.DMA,
        pltpu.SemaphoreType.DMA,
        pltpu.SemaphoreType.DMA,
    ],
)
def _agg_kernel(table, src_m, dst_m, zeros_hbm, out,
                acc, sidx, didx, b0, b1, b2, b3, b4, b5, dbuf,
                g0, g1, g2, g3, g4, g5, s0, s1, s2, s3, s4, s5):
    c = lax.axis_index("c")
    s = lax.axis_index("s")
    wid = c * NS + s
    accv = acc
    bufs = (b0, b1, b2, b3, b4, b5)
    gsems = (g0, g1, g2, g3, g4, g5)
    ssems = (s0, s1, s2, s3, s4, s5)
    RING = 6
    LAG = 3
    pltpu.sync_copy(zeros_hbm, dbuf)
    for k in range(5):
        chunk = s + k * NS

        @pl.when(chunk < 78)
        def _zero(chunk=chunk):
            pltpu.sync_copy(dbuf, acc.at[pl.ds(pl.multiple_of(chunk * ECHUNK, 8), ECHUNK)])

        @pl.when(chunk == 78)
        def _zero_tail(chunk=chunk):
            pltpu.sync_copy(dbuf.at[pl.ds(0, 16)],
                            acc.at[pl.ds(pl.multiple_of(chunk * ECHUNK, 8), 16)])

    plsc.subcore_barrier()
    base = wid * NCH
    pend_g = [None] * 6
    pend_s = [None] * 6
    pltpu.sync_copy(src_m.at[pl.ds(base, NCH)], sidx)
    pltpu.sync_copy(dst_m.at[pl.ds(base, NCH)], didx)
    for g in range(NCH):
        b = g % RING
        if pend_s[b] is not None:
            pend_s[b].wait()
            pend_s[b] = None
        pend_g[b] = pltpu.async_copy(table.at[sidx.at[g]], bufs[b], gsems[b])
        g2_ = g - LAG
        if g2_ >= 0:
            b2_ = g2_ % RING
            pend_g[b2_].wait()
            pend_s[b2_] = pltpu.async_copy(
                bufs[b2_], accv.at[didx.at[g2_]], ssems[b2_], add=True)
    for g2_ in range(NCH - LAG, NCH):
        b2_ = g2_ % RING
        pend_g[b2_].wait()
        pend_s[b2_] = pltpu.async_copy(
            bufs[b2_], accv.at[didx.at[g2_]], ssems[b2_], add=True)
    for b in range(6):
        if pend_s[b] is not None:
            pend_s[b].wait()
    @pl.when(wid < 4)
    def _tail():
        pltpu.sync_copy(src_m.at[NW * NCH + wid], sidx.at[0])
        pltpu.sync_copy(dst_m.at[NW * NCH + wid], didx.at[0])
        pltpu.async_copy(table.at[sidx.at[0]], b0, g0).wait()
        pltpu.async_copy(b0, accv.at[didx.at[0]], s0, add=True).wait()

    plsc.subcore_barrier()
    for k in range(5):
        chunk = s + k * NS

        @pl.when(chunk < 78)
        def _dump(chunk=chunk):
            pltpu.sync_copy(acc.at[pl.ds(pl.multiple_of(chunk * ECHUNK, 8), ECHUNK)], dbuf)
            pltpu.sync_copy(dbuf, out.at[pl.ds(pl.multiple_of(c * N + chunk * ECHUNK, 8), ECHUNK)])

        @pl.when(chunk == 78)
        def _dump_tail(chunk=chunk):
            pltpu.sync_copy(acc.at[pl.ds(pl.multiple_of(chunk * ECHUNK, 8), 16)], dbuf.at[pl.ds(0, 16)])
            pltpu.sync_copy(dbuf.at[pl.ds(0, 16)],
                            out.at[pl.ds(pl.multiple_of(c * N + chunk * ECHUNK, 8), 16)])


_BLK = 1000
_GRID = N // _BLK


def _norm(deg_ref):
    d = deg_ref[0] + deg_ref[1]
    return lax.rsqrt(jnp.maximum(d, 1.0))


def _pre_body(x_ref, w1_ref, ds_ref, t1_ref):
    xw = jnp.dot(x_ref[...], w1_ref[...], preferred_element_type=_f32)
    t1_ref[...] = (xw * _norm(ds_ref)).astype(_bf16)


def _mid_body(p_ref, dd_ref, ds_ref, b_ref, w_ref, h_ref, t_ref):
    agg = p_ref[0].astype(_f32) + p_ref[1].astype(_f32)
    y = agg * _norm(dd_ref) + b_ref[...]
    h = jnp.where(y > 0, y, jnp.exp(jnp.minimum(y, 0.0)) - 1.0)
    h_ref[...] = h
    hw = jnp.dot(h, w_ref[...], preferred_element_type=_f32)
    t_ref[...] = (hw * _norm(ds_ref)).astype(_bf16)


def _last_body(p_ref, dd_ref, b_ref, h_ref):
    agg = p_ref[0].astype(_f32) + p_ref[1].astype(_f32)
    y = agg * _norm(dd_ref) + b_ref[...]
    m = jnp.max(y, axis=1, keepdims=True)
    e = jnp.exp(y - m)
    h_ref[...] = e / jnp.sum(e, axis=1, keepdims=True)


def _row_spec():
    return pl.BlockSpec((_BLK, D), lambda i: (i, 0))


def _deg_spec():
    return pl.BlockSpec((2, _BLK, 1), lambda i: (0, i, 0))


def _tc_pre(x, w1, deg_s):
    return pl.pallas_call(
        _pre_body,
        grid=(_GRID,),
        in_specs=[
            _row_spec(),
            pl.BlockSpec((D, D), lambda i: (0, 0)),
            _deg_spec(),
        ],
        out_specs=_row_spec(),
        out_shape=jax.ShapeDtypeStruct((N, D), _bf16),
    )(x, w1, deg_s)


def _tc_mid(p, deg_d, deg_s, b, w_next):
    return pl.pallas_call(
        _mid_body,
        grid=(_GRID,),
        in_specs=[
            pl.BlockSpec((2, _BLK, D), lambda i: (0, i, 0)),
            _deg_spec(),
            _deg_spec(),
            pl.BlockSpec((1, D), lambda i: (0, 0)),
            pl.BlockSpec((D, D), lambda i: (0, 0)),
        ],
        out_specs=[_row_spec(), _row_spec()],
        out_shape=[jax.ShapeDtypeStruct((N, D), _f32),
                   jax.ShapeDtypeStruct((N, D), _bf16)],
    )(p, deg_d, deg_s, b, w_next)


def _tc_last(p, deg_d, b):
    return pl.pallas_call(
        _last_body,
        grid=(_GRID,),
        in_specs=[
            pl.BlockSpec((2, _BLK, D), lambda i: (0, i, 0)),
            _deg_spec(),
            pl.BlockSpec((1, D), lambda i: (0, 0)),
        ],
        out_specs=_row_spec(),
        out_shape=jax.ShapeDtypeStruct((N, D), _f32),
    )(p, deg_d, b)


def kernel(x, edge_index, W1, b1, W2, b2, W3, b3):
    src = edge_index[0].astype(jnp.int32).reshape(NW, ROWS, CHUNK)
    dst = edge_index[1].astype(jnp.int32).reshape(NW, ROWS, CHUNK)
    src_e = edge_index[0].astype(jnp.int32).reshape(E // ECHUNK, ECHUNK)
    dst_e = edge_index[1].astype(jnp.int32).reshape(E // ECHUNK, ECHUNK)
    ones_c = jnp.ones((CHUNK,), _f32)
    zeros_1k = jnp.zeros((1000,), _f32)
    zeros_rows = jnp.zeros((ECHUNK, D), _bf16)

    deg_s, deg_d = _deg_kernel(src, dst, ones_c, zeros_1k)
    deg_s = deg_s.reshape(NC, N, 1)
    deg_d = deg_d.reshape(NC, N, 1)
    table1 = _tc_pre(x, W1, deg_s)

    p1 = _agg_kernel(table1, src_e, dst_e, zeros_rows).reshape(NC, N, D)
    h1, table2 = _tc_mid(p1, deg_d, deg_s, b1.reshape(1, D), W2)

    p2 = _agg_kernel(table2, src_e, dst_e, zeros_rows).reshape(NC, N, D)
    h2, table3 = _tc_mid(p2, deg_d, deg_s, b2.reshape(1, D), W3)

    p3 = _agg_kernel(table3, src_e, dst_e, zeros_rows).reshape(NC, N, D)
    h3 = _tc_last(p3, deg_d, b3.reshape(1, D))

    return (h1.reshape(1, N, D), h2.reshape(1, N, D), h3.reshape(1, N, D))
